# baseline (speedup 1.0000x reference)
import functools

import jax
import jax.numpy as jnp
from jax import lax
from jax.experimental import pallas as pl
from jax.experimental.pallas import tpu as pltpu
from jax.experimental.pallas import tpu_sc as plsc

B = 4096
L = 200
D = 64
SCALE = 8.0
PAD = 128


def kernel(x, table, pos_encoding):
    info = plsc.get_sparse_core_info()
    nc, ns = info.num_cores, info.num_subcores
    nw = nc * ns
    b_per_w = B // nw

    mesh = plsc.VectorSubcoreMesh(core_axis_name="c", subcore_axis_name="s")

    @functools.partial(
        pl.kernel,
        mesh=mesh,
        out_type=jax.ShapeDtypeStruct((L, D, B), jnp.float32),
        scratch_types=[
            pltpu.VMEM((L, b_per_w), jnp.int32),
            pltpu.VMEM((L, D), jnp.float32),
            pltpu.VMEM((2, b_per_w, PAD), jnp.float32),
            pltpu.VMEM((2, D, b_per_w), jnp.float32),
            [pltpu.SemaphoreType.DMA] * 2,
            [pltpu.SemaphoreType.DMA] * 2,
        ],
        compiler_params=pltpu.CompilerParams(
            use_tc_tiling_on_sc=False, needs_layout_passes=False
        ),
    )
    def sc_kernel(xt_hbm, pos_hbm, t128_hbm, out_hbm,
                  idx_v, pos_v, emb_v, out_v, gsems, osems):
        wid = lax.axis_index("s") * nc + lax.axis_index("c")
        b0 = wid * b_per_w

        pltpu.sync_copy(xt_hbm.at[:, pl.ds(b0, b_per_w)], idx_v)
        pltpu.sync_copy(pos_hbm, pos_v)

        iota = lax.iota(jnp.int32, 16)
        d_idx = [iota + 16 * j for j in range(D // 16)]

        def start_gather(l, eb):
            pltpu.async_copy(
                t128_hbm.at[idx_v.at[l]], emb_v.at[eb], gsems[eb]
            )

        def wait_gather(eb):
            pltpu.make_async_copy(
                t128_hbm.at[idx_v.at[0]], emb_v.at[eb], gsems[eb]
            ).wait()

        def start_store(l, ob):
            pltpu.async_copy(
                out_v.at[ob], out_hbm.at[l, :, pl.ds(b0, b_per_w)], osems[ob]
            )

        def wait_store(ob):
            pltpu.make_async_copy(
                out_v.at[ob], out_hbm.at[0, :, pl.ds(b0, b_per_w)], osems[ob]
            ).wait()

        def compute(l, eb, ob):
            pv = [pos_v[l, pl.ds(16 * j, 16)] for j in range(D // 16)]

            @plsc.parallel_loop(0, b_per_w, step=1, unroll=4)
            def row_body(r):
                b_col = jnp.full((16,), r, jnp.int32)
                for j in range(D // 16):
                    e = emb_v[eb, r, pl.ds(16 * j, 16)]
                    plsc.store_scatter(
                        out_v.at[ob], [d_idx[j], b_col], e * SCALE + pv[j]
                    )

        start_gather(0, 0)

        def pair_body(k, carry):
            for u in range(2):
                l = k * 2 + u

                @pl.when(l < L - 1)
                def _():
                    start_gather(l + 1, 1 - u)

                wait_gather(u)

                @pl.when(l >= 2)
                def _():
                    wait_store(u)

                compute(l, u, u)
                start_store(l, u)
            return carry

        lax.fori_loop(0, L // 2, pair_body, 0)
        wait_store(0)
        wait_store(1)

    xt = x.T
    t128 = jnp.pad(table, ((0, 0), (0, PAD - D)))
    out_ldb = sc_kernel(xt, pos_encoding, t128)
    return out_ldb.transpose(2, 0, 1)


# speedup vs baseline: 1.4581x; 1.4581x over previous
import functools

import jax
import jax.numpy as jnp
from jax import lax
from jax.experimental import pallas as pl
from jax.experimental.pallas import tpu as pltpu
from jax.experimental.pallas import tpu_sc as plsc

B = 4096
L = 200
D = 64
SCALE = 8.0
PAD = 128


def kernel(x, table, pos_encoding):
    info = plsc.get_sparse_core_info()
    nc, ns = info.num_cores, info.num_subcores
    nw = nc * ns
    b_per_w = B // nw

    mesh = plsc.VectorSubcoreMesh(core_axis_name="c", subcore_axis_name="s")

    @functools.partial(
        pl.kernel,
        mesh=mesh,
        out_type=jax.ShapeDtypeStruct((L, D, B), jnp.float32),
        scratch_types=[
            pltpu.VMEM((L, b_per_w), jnp.int32),
            pltpu.VMEM((L, D), jnp.float32),
            pltpu.VMEM((2, b_per_w, PAD), jnp.float32),
            pltpu.VMEM((2, D, 133), jnp.float32),
            [pltpu.SemaphoreType.DMA] * 2,
            [pltpu.SemaphoreType.DMA] * 2,
        ],
        compiler_params=pltpu.CompilerParams(
            use_tc_tiling_on_sc=False, needs_layout_passes=False
        ),
    )
    def sc_kernel(xt_hbm, pos_hbm, t128_hbm, out_hbm,
                  idx_v, pos_v, emb_v, out_v, gsems, osems):
        wid = lax.axis_index("s") * nc + lax.axis_index("c")
        b0 = wid * b_per_w

        pltpu.sync_copy(xt_hbm.at[:, pl.ds(b0, b_per_w)], idx_v)
        pltpu.sync_copy(pos_hbm, pos_v)

        iota = lax.iota(jnp.int32, 16)
        d_idx = [iota + 16 * j for j in range(D // 16)]

        def start_gather(l, eb):
            pltpu.async_copy(
                t128_hbm.at[idx_v.at[l]], emb_v.at[eb], gsems[eb]
            )

        def wait_gather(eb):
            pltpu.make_async_copy(
                t128_hbm.at[idx_v.at[0]], emb_v.at[eb], gsems[eb]
            ).wait()

        def start_store(l, ob):
            pltpu.async_copy(
                out_v.at[ob, :, pl.ds(0, b_per_w)],
                out_hbm.at[l, :, pl.ds(b0, b_per_w)], osems[ob]
            )

        def wait_store(ob):
            pltpu.make_async_copy(
                out_v.at[ob, :, pl.ds(0, b_per_w)],
                out_hbm.at[0, :, pl.ds(b0, b_per_w)], osems[ob]
            ).wait()

        def compute(l, eb, ob):
            pv = [pos_v[l, pl.ds(16 * j, 16)] for j in range(D // 16)]

            @plsc.parallel_loop(0, b_per_w, step=1, unroll=4)
            def row_body(r):
                b_col = jnp.full((16,), r, jnp.int32)
                for j in range(D // 16):
                    e = emb_v[eb, r, pl.ds(16 * j, 16)]
                    plsc.store_scatter(
                        out_v.at[ob], [d_idx[j], b_col], e * SCALE + pv[j]
                    )

        start_gather(0, 0)

        def pair_body(k, carry):
            for u in range(2):
                l = k * 2 + u

                @pl.when(l < L - 1)
                def _():
                    start_gather(l + 1, 1 - u)

                wait_gather(u)

                @pl.when(l >= 2)
                def _():
                    wait_store(u)

                compute(l, u, u)
                start_store(l, u)
            return carry

        lax.fori_loop(0, L // 2, pair_body, 0)
        wait_store(0)
        wait_store(1)

    xt = x.T
    t128 = jnp.pad(table, ((0, 0), (0, PAD - D)))
    out_ldb = sc_kernel(xt, pos_encoding, t128)
    return out_ldb.transpose(2, 0, 1)


# speedup vs baseline: 1.8526x; 1.2705x over previous
import functools

import jax
import jax.numpy as jnp
from jax import lax
from jax.experimental import pallas as pl
from jax.experimental.pallas import tpu as pltpu
from jax.experimental.pallas import tpu_sc as plsc

B = 4096
L = 200
D = 64
SCALE = 8.0
PAD = 128
VOCAB = 1000000
_TC_CHUNK = 2000
_PITCH = 133


def _transpose_table(table_t):
    def body(t_ref, out_ref):
        out_ref[:, 0:D] = t_ref[...].T
        out_ref[:, D:PAD] = jnp.zeros((_TC_CHUNK, PAD - D), jnp.float32)

    return pl.pallas_call(
        body,
        grid=(VOCAB // _TC_CHUNK,),
        in_specs=[pl.BlockSpec((D, _TC_CHUNK), lambda i: (0, i))],
        out_specs=pl.BlockSpec((_TC_CHUNK, PAD), lambda i: (i, 0)),
        out_shape=jax.ShapeDtypeStruct((VOCAB, PAD), jnp.float32),
    )(table_t)


def kernel(x, table, pos_encoding):
    info = plsc.get_sparse_core_info()
    nc, ns = info.num_cores, info.num_subcores
    nw = nc * ns
    b_per_w = B // nw

    mesh = plsc.VectorSubcoreMesh(core_axis_name="c", subcore_axis_name="s")

    @functools.partial(
        pl.kernel,
        mesh=mesh,
        out_type=jax.ShapeDtypeStruct((L, 8, nw, 8, 128), jnp.float32),
        scratch_types=[
            pltpu.VMEM((L, b_per_w), jnp.int32),
            pltpu.VMEM((L, D), jnp.float32),
            pltpu.VMEM((2, b_per_w, PAD), jnp.float32),
            pltpu.VMEM((2, 8, 8, _PITCH), jnp.float32),
            [pltpu.SemaphoreType.DMA] * 2,
            [pltpu.SemaphoreType.DMA] * 2,
        ],
        compiler_params=pltpu.CompilerParams(
            use_tc_tiling_on_sc=False, needs_layout_passes=False
        ),
    )
    def sc_kernel(xt_hbm, pos_hbm, t128_hbm, out_hbm,
                  idx_v, pos_v, emb_v, out_v, gsems, osems):
        wid = lax.axis_index("s") * nc + lax.axis_index("c")
        b0 = wid * b_per_w

        pltpu.sync_copy(xt_hbm.at[:, pl.ds(b0, b_per_w)], idx_v)
        pltpu.sync_copy(pos_hbm, pos_v)

        iota = lax.iota(jnp.int32, 16)
        dt_idx = [iota // 8 + 2 * j for j in range(D // 16)]
        di_idx = iota % 8

        def start_gather(l, eb):
            pltpu.async_copy(
                t128_hbm.at[idx_v.at[l]], emb_v.at[eb], gsems[eb]
            )

        def wait_gather(eb):
            pltpu.make_async_copy(
                t128_hbm.at[idx_v.at[0]], emb_v.at[eb], gsems[eb]
            ).wait()

        def start_store(l, ob):
            pltpu.async_copy(
                out_v.at[ob, :, :, pl.ds(0, b_per_w)],
                out_hbm.at[l, :, wid], osems[ob]
            )

        def wait_store(ob):
            pltpu.make_async_copy(
                out_v.at[ob, :, :, pl.ds(0, b_per_w)],
                out_hbm.at[0, :, wid], osems[ob]
            ).wait()

        def compute(l, eb, ob):
            pv = [pos_v[l, pl.ds(16 * j, 16)] for j in range(D // 16)]

            @plsc.parallel_loop(0, b_per_w, step=1, unroll=4)
            def row_body(r):
                b_col = jnp.full((16,), r, jnp.int32)
                for j in range(D // 16):
                    e = emb_v[eb, r, pl.ds(16 * j, 16)]
                    plsc.store_scatter(
                        out_v.at[ob], [dt_idx[j], di_idx, b_col],
                        e * SCALE + pv[j],
                    )

        start_gather(0, 0)

        def pair_body(k, carry):
            for u in range(2):
                l = k * 2 + u

                @pl.when(l < L - 1)
                def _():
                    start_gather(l + 1, 1 - u)

                wait_gather(u)

                @pl.when(l >= 2)
                def _():
                    wait_store(u)

                compute(l, u, u)
                start_store(l, u)
            return carry

        lax.fori_loop(0, L // 2, pair_body, 0)
        wait_store(0)
        wait_store(1)

    xt = x.T
    t128 = jnp.pad(table, ((0, 0), (0, PAD - D)))
    out5 = sc_kernel(xt, pos_encoding, t128)
    return out5.transpose(2, 4, 0, 1, 3).reshape(B, L, D)


# speedup vs baseline: 1.9162x; 1.0344x over previous
import functools

import jax
import jax.numpy as jnp
from jax import lax
from jax.experimental import pallas as pl
from jax.experimental.pallas import tpu as pltpu
from jax.experimental.pallas import tpu_sc as plsc

B = 4096
L = 200
D = 64
SCALE = 8.0
PAD = 128
VOCAB = 1000000
_PITCH = 133


def kernel(x, table, pos_encoding):
    info = plsc.get_sparse_core_info()
    nc, ns = info.num_cores, info.num_subcores
    nw = nc * ns
    b_per_w = B // nw

    mesh = plsc.VectorSubcoreMesh(core_axis_name="c", subcore_axis_name="s")

    @functools.partial(
        pl.kernel,
        mesh=mesh,
        out_type=jax.ShapeDtypeStruct((L, 8, nw, 8, 128), jnp.float32),
        scratch_types=[
            pltpu.VMEM((L, b_per_w), jnp.int32),
            pltpu.VMEM((L, D), jnp.float32),
            pltpu.VMEM((4, b_per_w, PAD), jnp.float32),
            pltpu.VMEM((2, 8, 8, _PITCH), jnp.float32),
            [pltpu.SemaphoreType.DMA] * 4,
            [pltpu.SemaphoreType.DMA] * 2,
        ],
        compiler_params=pltpu.CompilerParams(
            use_tc_tiling_on_sc=False, needs_layout_passes=False
        ),
    )
    def sc_kernel(xt_hbm, pos_hbm, t128_hbm, out_hbm,
                  idx_v, pos_v, emb_v, out_v, gsems, osems):
        wid = lax.axis_index("s") * nc + lax.axis_index("c")
        b0 = wid * b_per_w

        pltpu.sync_copy(xt_hbm.at[:, pl.ds(b0, b_per_w)], idx_v)
        pltpu.sync_copy(pos_hbm, pos_v)

        iota = lax.iota(jnp.int32, 16)
        dt_idx = [iota // 8 + 2 * j for j in range(D // 16)]
        di_idx = iota % 8

        def start_gather(l, eb):
            pltpu.async_copy(
                t128_hbm.at[idx_v.at[l]], emb_v.at[eb], gsems[eb]
            )

        def wait_gather(eb):
            pltpu.make_async_copy(
                t128_hbm.at[idx_v.at[0]], emb_v.at[eb], gsems[eb]
            ).wait()

        def start_store(l, ob):
            pltpu.async_copy(
                out_v.at[ob, :, :, pl.ds(0, b_per_w)],
                out_hbm.at[l, :, wid], osems[ob]
            )

        def wait_store(ob):
            pltpu.make_async_copy(
                out_v.at[ob, :, :, pl.ds(0, b_per_w)],
                out_hbm.at[0, :, wid], osems[ob]
            ).wait()

        def compute(l, eb, ob):
            pv = [pos_v[l, pl.ds(16 * j, 16)] for j in range(D // 16)]

            @plsc.parallel_loop(0, b_per_w, step=1, unroll=4)
            def row_body(r):
                b_col = jnp.full((16,), r, jnp.int32)
                for j in range(D // 16):
                    e = emb_v[eb, r, pl.ds(16 * j, 16)]
                    plsc.store_scatter(
                        out_v.at[ob], [dt_idx[j], di_idx, b_col],
                        e * SCALE + pv[j],
                    )

        for b in range(4):
            start_gather(b, b)

        def quad_body(k, carry):
            for u in range(4):
                l = k * 4 + u
                wait_gather(u)

                @pl.when(l >= 2)
                def _():
                    wait_store(u % 2)

                compute(l, u, u % 2)
                start_store(l, u % 2)

                @pl.when(l + 4 < L)
                def _():
                    start_gather(l + 4, u)
            return carry

        lax.fori_loop(0, L // 4, quad_body, 0)
        wait_store(0)
        wait_store(1)

    xt = x.T
    t128 = jnp.pad(table, ((0, 0), (0, PAD - D)))
    out5 = sc_kernel(xt, pos_encoding, t128)
    return out5.transpose(2, 4, 0, 1, 3).reshape(B, L, D)
